# Initial kernel scaffold; baseline (speedup 1.0000x reference)
import jax
import jax.numpy as jnp
from jax.experimental import pallas as pl


def kernel(mov, ref, defgrad):
    raise NotImplementedError("write your pallas kernel here")


# speedup vs baseline: 1.2161x; 1.2161x over previous
import functools

import jax
import jax.numpy as jnp
from jax import lax
from jax.experimental import pallas as pl
from jax.experimental.pallas import tpu as pltpu
from jax.experimental.pallas import tpu_sc as plsc

_MAXGRAD = 2.0
_B = 2
_D = 128
_D2 = _D * _D
_D3 = _D * _D * _D
_N = _B * _D3


def _logistic(x):
    c = _MAXGRAD
    return c / (1.0 + (c - 1.0) * jnp.exp(-x))


def _grids_body(d0r, d1r, d2r, s0r, n0r, i0r, s1r, n1r, i1r, s2r, n2r, i2r):
    f32 = jnp.float32
    r = lax.broadcasted_iota(jnp.int32, (_D, _D), 0)
    cidx = lax.broadcasted_iota(jnp.int32, (_D, _D), 1)
    ltri = (cidx <= r).astype(f32)
    utri = (r <= cidx).astype(f32)

    def mm(a, b):
        return lax.dot_general(
            a, b, (((1,), (0,)), ((), ())),
            precision=lax.Precision.HIGHEST,
            preferred_element_type=f32)

    a = _logistic(d0r[0])
    s = mm(ltri, a.reshape(_D, -1)).reshape(a.shape)
    first = s[0:1]
    last = s[_D - 1:_D]
    n = (_D - 1.0) * (s - first) / (last - first + 1e-7)
    s0r[...] = (s - 1.0)[None]
    n0r[...] = n[None]
    base = lax.broadcasted_iota(jnp.int32, a.shape, 0).astype(f32)
    i0r[...] = (2.0 * base - n)[None]

    a = _logistic(d1r[0])
    base = lax.broadcasted_iota(jnp.int32, (_D, _D), 0).astype(f32)
    for i in range(a.shape[0]):
        s = mm(ltri, a[i])
        first = s[0:1, :]
        last = s[_D - 1:_D, :]
        n = (_D - 1.0) * (s - first) / (last - first + 1e-7)
        s1r[0, i] = s - 1.0
        n1r[0, i] = n
        i1r[0, i] = 2.0 * base - n

    a = _logistic(d2r[0])
    base = lax.broadcasted_iota(jnp.int32, (_D, _D), 1).astype(f32)
    for i in range(a.shape[0]):
        s = mm(a[i], utri)
        first = s[:, 0:1]
        last = s[:, _D - 1:_D]
        n = (_D - 1.0) * (s - first) / (last - first + 1e-7)
        s2r[0, i] = s - 1.0
        n2r[0, i] = n
        i2r[0, i] = 2.0 * base - n


def _grids(d0, d1, d2, interpret=False):
    xspec = pl.BlockSpec((1, _D, 16, _D), lambda b, j: (b, 0, j, 0))
    yspec = pl.BlockSpec((1, 16, _D, _D), lambda b, j: (b, j, 0, 0))
    shp = jax.ShapeDtypeStruct((_B, _D, _D, _D), jnp.float32)
    return pl.pallas_call(
        _grids_body,
        grid=(_B, _D // 16),
        in_specs=[xspec, yspec, yspec],
        out_specs=[xspec, xspec, xspec,
                   yspec, yspec, yspec,
                   yspec, yspec, yspec],
        out_shape=[shp] * 9,
        interpret=interpret,
    )(d0, d1, d2)


_NW = 32
_NPW = _N // _NW
_CK = 2048
_NCHUNK = _NPW // _CK
_ROWS = _CK // _D


def _resample_body(vol, cx, cy, cz, out, cxb, cyb, czb, xdb, ydb, zdb,
                   outb, idxb, valb, sem):
    wid = lax.axis_index("s") * 2 + lax.axis_index("c")
    batch = wid // 16
    base_pt = wid * _NPW
    vbase = batch * _D3

    def chunk(t, _):
        off = base_pt + t * _CK
        pltpu.sync_copy(cx.at[pl.ds(off, _CK)], cxb)
        pltpu.sync_copy(cy.at[pl.ds(off, _CK)], cyb)
        pltpu.sync_copy(cz.at[pl.ds(off, _CK)], czb)

        def prep(i, _):
            row = i >> 3
            col = (i & 7) * 16
            sl = pl.ds(i * 16, 16)
            x = jnp.clip(cxb[sl], 0.0, _D - 1.0)
            y = jnp.clip(cyb[sl], 0.0, _D - 1.0)
            z = jnp.clip(czb[sl], 0.0, _D - 1.0)
            x0 = jnp.minimum(x.astype(jnp.int32), _D - 2)
            y0 = jnp.minimum(y.astype(jnp.int32), _D - 2)
            z0 = jnp.minimum(z.astype(jnp.int32), _D - 2)
            xdb[sl] = x - x0.astype(jnp.float32)
            ydb[sl] = y - y0.astype(jnp.float32)
            zdb[sl] = z - z0.astype(jnp.float32)
            lin = vbase + x0 * _D2 + y0 * _D + z0
            for c in range(8):
                dx, dy, dz = (c >> 2) & 1, (c >> 1) & 1, c & 1
                idxb[c, row, pl.ds(col, 16)] = lin + dx * _D2 + dy * _D + dz
            return 0

        lax.fori_loop(0, _CK // 16, prep, 0)

        copies = []
        for c in range(8):
            for j in range(_ROWS):
                copies.append(pltpu.async_copy(
                    vol.at[idxb.at[c, j]], valb.at[c, j], sem))
        for cp in copies:
            cp.wait()

        def blend(i, _):
            row = i >> 3
            col = (i & 7) * 16
            sl = pl.ds(i * 16, 16)
            cs = pl.ds(col, 16)
            xd = xdb[sl]
            yd = ydb[sl]
            zd = zdb[sl]
            v0 = valb[0, row, cs]
            v1 = valb[1, row, cs]
            v2 = valb[2, row, cs]
            v3 = valb[3, row, cs]
            v4 = valb[4, row, cs]
            v5 = valb[5, row, cs]
            v6 = valb[6, row, cs]
            v7 = valb[7, row, cs]
            z00 = v0 + zd * (v1 - v0)
            z01 = v2 + zd * (v3 - v2)
            z10 = v4 + zd * (v5 - v4)
            z11 = v6 + zd * (v7 - v6)
            y0v = z00 + yd * (z01 - z00)
            y1v = z10 + yd * (z11 - z10)
            outb[sl] = y0v + xd * (y1v - y0v)
            return 0

        lax.fori_loop(0, _CK // 16, blend, 0)
        pltpu.sync_copy(outb, out.at[pl.ds(off, _CK)])
        return 0

    lax.fori_loop(0, _NCHUNK, chunk, 0)


@functools.partial(jax.jit, static_argnames=("interpret",))
def _resample(vol, cx, cy, cz, interpret=False):
    mesh = plsc.VectorSubcoreMesh(
        core_axis_name="c", subcore_axis_name="s", num_cores=2)
    return pl.kernel(
        _resample_body,
        out_type=jax.ShapeDtypeStruct((_N,), jnp.float32),
        mesh=mesh,
        scratch_types=[
            pltpu.VMEM((_CK,), jnp.float32),
            pltpu.VMEM((_CK,), jnp.float32),
            pltpu.VMEM((_CK,), jnp.float32),
            pltpu.VMEM((_CK,), jnp.float32),
            pltpu.VMEM((_CK,), jnp.float32),
            pltpu.VMEM((_CK,), jnp.float32),
            pltpu.VMEM((_CK,), jnp.float32),
            pltpu.VMEM((8, _ROWS, _D), jnp.int32),
            pltpu.VMEM((8, _ROWS, _D), jnp.float32),
            pltpu.SemaphoreType.DMA,
        ],
        interpret=interpret,
    )(vol, cx, cy, cz)


def kernel(mov, ref, defgrad):
    d0 = defgrad[..., 0]
    d1 = defgrad[..., 1]
    d2 = defgrad[..., 2]
    s0, n0, i0, s1, n1, i1, s2, n2, i2 = _grids(d0, d1, d2)

    norm = jnp.stack([n0, n1, n2], axis=-1)
    inverse = jnp.stack([i0, i1, i2], axis=-1)

    mov_def = _resample(mov.reshape(-1), s0.reshape(-1), s1.reshape(-1),
                        s2.reshape(-1))
    ref_def = _resample(ref.reshape(-1), i0.reshape(-1), i1.reshape(-1),
                        i2.reshape(-1))

    out_shape = (_B, _D, _D, _D, 1)
    return (mov_def.reshape(out_shape), ref_def.reshape(out_shape),
            norm, inverse)


# speedup vs baseline: 2.2382x; 1.8405x over previous
import functools

import jax
import jax.numpy as jnp
from jax import lax
from jax.experimental import pallas as pl
from jax.experimental.pallas import tpu as pltpu
from jax.experimental.pallas import tpu_sc as plsc

_MAXGRAD = 2.0
_B = 2
_D = 128
_D2 = _D * _D
_D3 = _D * _D * _D
_N = _B * _D3


def _logistic(x):
    c = _MAXGRAD
    return c / (1.0 + (c - 1.0) * jnp.exp(-x))


def _grids_body(d0r, d1r, d2r, s0r, n0r, i0r, s1r, n1r, i1r, s2r, n2r, i2r):
    f32 = jnp.float32
    r = lax.broadcasted_iota(jnp.int32, (_D, _D), 0)
    cidx = lax.broadcasted_iota(jnp.int32, (_D, _D), 1)
    ltri = (cidx <= r).astype(f32)
    utri = (r <= cidx).astype(f32)

    def mm(a, b):
        return lax.dot_general(
            a, b, (((1,), (0,)), ((), ())),
            precision=lax.Precision.HIGHEST,
            preferred_element_type=f32)

    a = _logistic(d0r[0])
    s = mm(ltri, a.reshape(_D, -1)).reshape(a.shape)
    first = s[0:1]
    last = s[_D - 1:_D]
    n = (_D - 1.0) * (s - first) / (last - first + 1e-7)
    s0r[...] = (s - 1.0)[None]
    n0r[...] = n[None]
    base = lax.broadcasted_iota(jnp.int32, a.shape, 0).astype(f32)
    i0r[...] = (2.0 * base - n)[None]

    a = _logistic(d1r[0])
    base = lax.broadcasted_iota(jnp.int32, (_D, _D), 0).astype(f32)
    for i in range(a.shape[0]):
        s = mm(ltri, a[i])
        first = s[0:1, :]
        last = s[_D - 1:_D, :]
        n = (_D - 1.0) * (s - first) / (last - first + 1e-7)
        s1r[0, i] = s - 1.0
        n1r[0, i] = n
        i1r[0, i] = 2.0 * base - n

    a = _logistic(d2r[0])
    base = lax.broadcasted_iota(jnp.int32, (_D, _D), 1).astype(f32)
    for i in range(a.shape[0]):
        s = mm(a[i], utri)
        first = s[:, 0:1]
        last = s[:, _D - 1:_D]
        n = (_D - 1.0) * (s - first) / (last - first + 1e-7)
        s2r[0, i] = s - 1.0
        n2r[0, i] = n
        i2r[0, i] = 2.0 * base - n


def _grids(d0, d1, d2, interpret=False):
    xspec = pl.BlockSpec((1, _D, 16, _D), lambda b, j: (b, 0, j, 0))
    yspec = pl.BlockSpec((1, 16, _D, _D), lambda b, j: (b, j, 0, 0))
    shp = jax.ShapeDtypeStruct((_B, _D, _D, _D), jnp.float32)
    return pl.pallas_call(
        _grids_body,
        grid=(_B, _D // 16),
        in_specs=[xspec, yspec, yspec],
        out_specs=[xspec, xspec, xspec,
                   yspec, yspec, yspec,
                   yspec, yspec, yspec],
        out_shape=[shp] * 9,
        interpret=interpret,
    )(d0, d1, d2)


_TAB = 64 * _D2


def _pp_body(cur_r, o_r):
    a = cur_r[0].reshape(8, 2, _D, _D)
    lo = lax.bitcast_convert_type(
        a[:, 0].astype(jnp.bfloat16), jnp.uint16).astype(jnp.uint32)
    hi = lax.bitcast_convert_type(
        a[:, 1].astype(jnp.bfloat16), jnp.uint16).astype(jnp.uint32)
    w = lax.bitcast_convert_type(lo | (hi << 16), jnp.int32)
    o_r[...] = w.reshape(8 * _D, _D)[None]


def _pppack(imt):
    return pl.pallas_call(
        _pp_body,
        grid=(_B, _D // 16),
        in_specs=[pl.BlockSpec((1, 16, _D, _D), lambda b, z: (b, z, 0, 0))],
        out_specs=pl.BlockSpec((1, 8 * _D, _D), lambda b, z: (b, z, 0)),
        out_shape=jax.ShapeDtypeStruct((_B, 64 * _D, _D), jnp.int32),
    )(imt).reshape(_B * _TAB)


_NW = 32
_NPW = _N // _NW
_CK = 1024
_NCHUNK = _NPW // _CK
_ROWS = _CK // _D
_STG = _TAB // 16
_STH = 8192


def _resample_body(pp, cx, cy, cz, out, cxb, cyb, czb, xdb, ydb, zdb, pob,
                   outb, idx, val, stb, tab, sem):
    cid = lax.axis_index("c")
    sid = lax.axis_index("s")
    base_pt = (cid * 16 + sid) * _NPW

    for h in range(_STG // _STH):
        soff = sid * _STG + h * _STH
        pltpu.sync_copy(pp.at[pl.ds(cid * _TAB + soff, _STH)], stb)
        pltpu.sync_copy(stb, tab.at[pl.ds(soff, _STH)])
    plsc.subcore_barrier()

    def chunk(t, _):
        off = base_pt + t * _CK
        pltpu.sync_copy(cx.at[pl.ds(off, _CK)], cxb)
        pltpu.sync_copy(cy.at[pl.ds(off, _CK)], cyb)
        pltpu.sync_copy(cz.at[pl.ds(off, _CK)], czb)

        def prep(i, _):
            row = i >> 3
            col = (i & 7) * 16
            sl = pl.ds(i * 16, 16)
            x = jnp.clip(cxb[sl], 0.0, _D - 1.0)
            y = jnp.clip(cyb[sl], 0.0, _D - 1.0)
            z = jnp.clip(czb[sl], 0.0, _D - 1.0)
            x0 = jnp.minimum(x.astype(jnp.int32), _D - 2)
            y0 = jnp.minimum(y.astype(jnp.int32), _D - 2)
            z0 = jnp.minimum(z.astype(jnp.int32), _D - 2)
            xdb[sl] = x - x0.astype(jnp.float32)
            ydb[sl] = y - y0.astype(jnp.float32)
            zdb[sl] = z - z0.astype(jnp.float32)
            podd = z0 & 1
            pob[sl] = podd
            v = (z0 >> 1) * _D2 + y0 * _D + x0
            vb = v + podd * _D2
            cs = pl.ds(col, 16)
            idx[0, row, cs] = v
            idx[1, row, cs] = vb
            idx[2, row, cs] = v + 1
            idx[3, row, cs] = vb + 1
            idx[4, row, cs] = v + _D
            idx[5, row, cs] = vb + _D
            idx[6, row, cs] = v + _D + 1
            idx[7, row, cs] = vb + _D + 1
            return 0

        lax.fori_loop(0, _CK // 16, prep, 0)

        copies = []
        for g in range(8):
            for j in range(_ROWS):
                copies.append(pltpu.async_copy(
                    tab.at[idx.at[g, j]], val.at[g, j], sem))
        for cp in copies:
            cp.wait()

        def blend(i, _):
            row = i >> 3
            cs = pl.ds((i & 7) * 16, 16)
            sl = pl.ds(i * 16, 16)
            xd = xdb[sl]
            yd = ydb[sl]
            zd = zdb[sl]
            odd = pob[sl] == 1

            def zmix(g):
                wa = val[2 * g, row, cs]
                wb = val[2 * g + 1, row, cs]
                lo1 = lax.bitcast_convert_type(wa << 16, jnp.float32)
                hi1 = lax.bitcast_convert_type(wa & jnp.int32(-65536),
                                               jnp.float32)
                lo2 = lax.bitcast_convert_type(wb << 16, jnp.float32)
                hi2 = lax.bitcast_convert_type(wb & jnp.int32(-65536),
                                               jnp.float32)
                vz0 = jnp.where(odd, hi1, lo1)
                vz1 = jnp.where(odd, lo2, hi2)
                return vz0 + zd * (vz1 - vz0)

            c00 = zmix(0)
            c01 = zmix(1)
            c10 = zmix(2)
            c11 = zmix(3)
            r0 = c00 + xd * (c01 - c00)
            r1 = c10 + xd * (c11 - c10)
            outb[sl] = r0 + yd * (r1 - r0)
            return 0

        lax.fori_loop(0, _CK // 16, blend, 0)
        pltpu.sync_copy(outb, out.at[pl.ds(off, _CK)])
        return 0

    lax.fori_loop(0, _NCHUNK, chunk, 0)


@functools.partial(jax.jit, static_argnames=("interpret",))
def _resample(pp, cx, cy, cz, interpret=False):
    mesh = plsc.VectorSubcoreMesh(
        core_axis_name="c", subcore_axis_name="s", num_cores=2)
    return pl.kernel(
        _resample_body,
        out_type=jax.ShapeDtypeStruct((_N,), jnp.float32),
        mesh=mesh,
        scratch_types=[
            pltpu.VMEM((_CK,), jnp.float32),
            pltpu.VMEM((_CK,), jnp.float32),
            pltpu.VMEM((_CK,), jnp.float32),
            pltpu.VMEM((_CK,), jnp.float32),
            pltpu.VMEM((_CK,), jnp.float32),
            pltpu.VMEM((_CK,), jnp.float32),
            pltpu.VMEM((_CK,), jnp.int32),
            pltpu.VMEM((_CK,), jnp.float32),
            pltpu.VMEM((8, _ROWS, _D), jnp.int32),
            pltpu.VMEM((8, _ROWS, _D), jnp.int32),
            pltpu.VMEM((_STH,), jnp.int32),
            pltpu.VMEM_SHARED((_TAB,), jnp.int32),
            pltpu.SemaphoreType.DMA,
        ],
        interpret=interpret,
    )(pp, cx, cy, cz)


def kernel(mov, ref, defgrad):
    d0 = defgrad[..., 0]
    d1 = defgrad[..., 1]
    d2 = defgrad[..., 2]
    s0, n0, i0, s1, n1, i1, s2, n2, i2 = _grids(d0, d1, d2)

    norm = jnp.stack([n0, n1, n2], axis=-1)
    inverse = jnp.stack([i0, i1, i2], axis=-1)

    mov_t = jnp.transpose(mov.reshape(_B, _D, _D, _D), (0, 3, 2, 1))
    ref_t = jnp.transpose(ref.reshape(_B, _D, _D, _D), (0, 3, 2, 1))
    mov_pp = _pppack(mov_t)
    ref_pp = _pppack(ref_t)

    mov_def = _resample(mov_pp, s0.reshape(-1), s1.reshape(-1),
                        s2.reshape(-1))
    ref_def = _resample(ref_pp, i0.reshape(-1), i1.reshape(-1),
                        i2.reshape(-1))

    out_shape = (_B, _D, _D, _D, 1)
    return (mov_def.reshape(out_shape), ref_def.reshape(out_shape),
            norm, inverse)


# speedup vs baseline: 3.0456x; 1.3607x over previous
import functools

import jax
import jax.numpy as jnp
from jax import lax
from jax.experimental import pallas as pl
from jax.experimental.pallas import tpu as pltpu
from jax.experimental.pallas import tpu_sc as plsc

_MAXGRAD = 2.0
_B = 2
_D = 128
_D2 = _D * _D
_D3 = _D * _D * _D
_N = _B * _D3


def _logistic(x):
    c = _MAXGRAD
    return c / (1.0 + (c - 1.0) * jnp.exp(-x))


def _grids_body(d0r, d1r, d2r, s0r, n0r, i0r, s1r, n1r, i1r, s2r, n2r, i2r):
    f32 = jnp.float32
    r = lax.broadcasted_iota(jnp.int32, (_D, _D), 0)
    cidx = lax.broadcasted_iota(jnp.int32, (_D, _D), 1)
    ltri = (cidx <= r).astype(f32)
    utri = (r <= cidx).astype(f32)

    def mm(a, b):
        return lax.dot_general(
            a, b, (((1,), (0,)), ((), ())),
            precision=lax.Precision.HIGHEST,
            preferred_element_type=f32)

    a = _logistic(d0r[0])
    s = mm(ltri, a.reshape(_D, -1)).reshape(a.shape)
    first = s[0:1]
    last = s[_D - 1:_D]
    n = (_D - 1.0) * (s - first) / (last - first + 1e-7)
    s0r[...] = (s - 1.0)[None]
    n0r[...] = n[None]
    base = lax.broadcasted_iota(jnp.int32, a.shape, 0).astype(f32)
    i0r[...] = (2.0 * base - n)[None]

    a = _logistic(d1r[0])
    base = lax.broadcasted_iota(jnp.int32, (_D, _D), 0).astype(f32)
    for i in range(a.shape[0]):
        s = mm(ltri, a[i])
        first = s[0:1, :]
        last = s[_D - 1:_D, :]
        n = (_D - 1.0) * (s - first) / (last - first + 1e-7)
        s1r[0, i] = s - 1.0
        n1r[0, i] = n
        i1r[0, i] = 2.0 * base - n

    a = _logistic(d2r[0])
    base = lax.broadcasted_iota(jnp.int32, (_D, _D), 1).astype(f32)
    for i in range(a.shape[0]):
        s = mm(a[i], utri)
        first = s[:, 0:1]
        last = s[:, _D - 1:_D]
        n = (_D - 1.0) * (s - first) / (last - first + 1e-7)
        s2r[0, i] = s - 1.0
        n2r[0, i] = n
        i2r[0, i] = 2.0 * base - n


def _grids(d0, d1, d2, interpret=False):
    xspec = pl.BlockSpec((1, _D, 16, _D), lambda b, j: (b, 0, j, 0))
    yspec = pl.BlockSpec((1, 16, _D, _D), lambda b, j: (b, j, 0, 0))
    shp = jax.ShapeDtypeStruct((_B, _D, _D, _D), jnp.float32)
    return pl.pallas_call(
        _grids_body,
        grid=(_B, _D // 16),
        in_specs=[xspec, yspec, yspec],
        out_specs=[xspec, xspec, xspec,
                   yspec, yspec, yspec,
                   yspec, yspec, yspec],
        out_shape=[shp] * 9,
        interpret=interpret,
    )(d0, d1, d2)


_TAB = 64 * _D2


def _pp_body(cur_r, o_r):
    a = cur_r[0].reshape(8, 2, _D, _D)
    lo = lax.bitcast_convert_type(
        a[:, 0].astype(jnp.bfloat16), jnp.uint16).astype(jnp.uint32)
    hi = lax.bitcast_convert_type(
        a[:, 1].astype(jnp.bfloat16), jnp.uint16).astype(jnp.uint32)
    w = lax.bitcast_convert_type(lo | (hi << 16), jnp.int32)
    o_r[...] = w.reshape(8 * _D, _D)[None]


def _pppack(imt):
    return pl.pallas_call(
        _pp_body,
        grid=(_B, _D // 16),
        in_specs=[pl.BlockSpec((1, 16, _D, _D), lambda b, z: (b, z, 0, 0))],
        out_specs=pl.BlockSpec((1, 8 * _D, _D), lambda b, z: (b, z, 0)),
        out_shape=jax.ShapeDtypeStruct((_B, 64 * _D, _D), jnp.int32),
    )(imt).reshape(_B * _TAB)


_NW = 32
_NPW = _N // _NW
_CK = 2048
_NCHUNK = _NPW // _CK
_ROWS = _CK // _D
_STG = _TAB // 16
_STH = 8192


def _resample_body(pp, cx, cy, cz, out, cxb, cyb, czb, xdb, ydb, zdb, pob,
                   outb, i0, i1, i2, i3, i4, i5, i6, i7,
                   v0, v1, v2, v3, v4, v5, v6, v7, stb, tab, sem):
    cid = lax.axis_index("c")
    sid = lax.axis_index("s")
    base_pt = (cid * 16 + sid) * _NPW

    for h in range(_STG // _STH):
        soff = sid * _STG + h * _STH
        pltpu.sync_copy(pp.at[pl.ds(cid * _TAB + soff, _STH)], stb)
        pltpu.sync_copy(stb, tab.at[pl.ds(soff, _STH)])
    plsc.subcore_barrier()

    def chunk(t, _):
        off = base_pt + t * _CK
        pltpu.sync_copy(cx.at[pl.ds(off, _CK)], cxb)
        pltpu.sync_copy(cy.at[pl.ds(off, _CK)], cyb)
        pltpu.sync_copy(cz.at[pl.ds(off, _CK)], czb)

        def prep(i, _):
            sl = pl.ds(i * 16, 16)
            x = jnp.clip(cxb[sl], 0.0, _D - 1.0)
            y = jnp.clip(cyb[sl], 0.0, _D - 1.0)
            z = jnp.clip(czb[sl], 0.0, _D - 1.0)
            x0 = jnp.minimum(x.astype(jnp.int32), _D - 2)
            y0 = jnp.minimum(y.astype(jnp.int32), _D - 2)
            z0 = jnp.minimum(z.astype(jnp.int32), _D - 2)
            xdb[sl] = x - x0.astype(jnp.float32)
            ydb[sl] = y - y0.astype(jnp.float32)
            zdb[sl] = z - z0.astype(jnp.float32)
            podd = z0 & 1
            pob[sl] = podd
            v = (z0 >> 1) * _D2 + y0 * _D + x0
            vb = v + podd * _D2
            i0[sl] = v
            i1[sl] = vb
            i2[sl] = v + 1
            i3[sl] = vb + 1
            i4[sl] = v + _D
            i5[sl] = vb + _D
            i6[sl] = v + _D + 1
            i7[sl] = vb + _D + 1
            return 0

        lax.fori_loop(0, _CK // 16, prep, 0)

        copies = []
        for ib, vbuf in ((i0, v0), (i1, v1), (i2, v2), (i3, v3),
                         (i4, v4), (i5, v5), (i6, v6), (i7, v7)):
            copies.append(pltpu.async_copy(tab.at[ib], vbuf, sem))
        for cp in copies:
            cp.wait()

        def blend(i, _):
            sl = pl.ds(i * 16, 16)
            xd = xdb[sl]
            yd = ydb[sl]
            zd = zdb[sl]
            odd = pob[sl] == 1

            def zmix(va, vbb):
                wa = va[sl]
                wb = vbb[sl]
                lo1 = lax.bitcast_convert_type(wa << 16, jnp.float32)
                hi1 = lax.bitcast_convert_type(wa & jnp.int32(-65536),
                                               jnp.float32)
                lo2 = lax.bitcast_convert_type(wb << 16, jnp.float32)
                hi2 = lax.bitcast_convert_type(wb & jnp.int32(-65536),
                                               jnp.float32)
                vz0 = jnp.where(odd, hi1, lo1)
                vz1 = jnp.where(odd, lo2, hi2)
                return vz0 + zd * (vz1 - vz0)

            c00 = zmix(v0, v1)
            c01 = zmix(v2, v3)
            c10 = zmix(v4, v5)
            c11 = zmix(v6, v7)
            r0 = c00 + xd * (c01 - c00)
            r1 = c10 + xd * (c11 - c10)
            outb[sl] = r0 + yd * (r1 - r0)
            return 0

        lax.fori_loop(0, _CK // 16, blend, 0)
        pltpu.sync_copy(outb, out.at[pl.ds(off, _CK)])
        return 0

    lax.fori_loop(0, _NCHUNK, chunk, 0)


@functools.partial(jax.jit, static_argnames=("interpret",))
def _resample(pp, cx, cy, cz, interpret=False):
    mesh = plsc.VectorSubcoreMesh(
        core_axis_name="c", subcore_axis_name="s", num_cores=2)
    return pl.kernel(
        _resample_body,
        out_type=jax.ShapeDtypeStruct((_N,), jnp.float32),
        mesh=mesh,
        scratch_types=[
            pltpu.VMEM((_CK,), jnp.float32),
            pltpu.VMEM((_CK,), jnp.float32),
            pltpu.VMEM((_CK,), jnp.float32),
            pltpu.VMEM((_CK,), jnp.float32),
            pltpu.VMEM((_CK,), jnp.float32),
            pltpu.VMEM((_CK,), jnp.float32),
            pltpu.VMEM((_CK,), jnp.int32),
            pltpu.VMEM((_CK,), jnp.float32),
        ] + [pltpu.VMEM((_CK,), jnp.int32)] * 16 + [
            pltpu.VMEM((_STH,), jnp.int32),
            pltpu.VMEM_SHARED((_TAB,), jnp.int32),
            pltpu.SemaphoreType.DMA,
        ],
        interpret=interpret,
    )(pp, cx, cy, cz)


def kernel(mov, ref, defgrad):
    d0 = defgrad[..., 0]
    d1 = defgrad[..., 1]
    d2 = defgrad[..., 2]
    s0, n0, i0, s1, n1, i1, s2, n2, i2 = _grids(d0, d1, d2)

    norm = jnp.stack([n0, n1, n2], axis=-1)
    inverse = jnp.stack([i0, i1, i2], axis=-1)

    mov_t = jnp.transpose(mov.reshape(_B, _D, _D, _D), (0, 3, 2, 1))
    ref_t = jnp.transpose(ref.reshape(_B, _D, _D, _D), (0, 3, 2, 1))
    mov_pp = _pppack(mov_t)
    ref_pp = _pppack(ref_t)

    mov_def = _resample(mov_pp, s0.reshape(-1), s1.reshape(-1),
                        s2.reshape(-1))
    ref_def = _resample(ref_pp, i0.reshape(-1), i1.reshape(-1),
                        i2.reshape(-1))

    out_shape = (_B, _D, _D, _D, 1)
    return (mov_def.reshape(out_shape), ref_def.reshape(out_shape),
            norm, inverse)


# speedup vs baseline: 4.5994x; 1.5102x over previous
import functools

import jax
import jax.numpy as jnp
from jax import lax
from jax.experimental import pallas as pl
from jax.experimental.pallas import tpu as pltpu
from jax.experimental.pallas import tpu_sc as plsc

_MAXGRAD = 2.0
_B = 2
_D = 128
_D2 = _D * _D
_D3 = _D * _D * _D
_N = _B * _D3


def _logistic(x):
    c = _MAXGRAD
    return c / (1.0 + (c - 1.0) * jnp.exp(-x))


def _grids_body(d0r, d1r, d2r, s0r, n0r, i0r, s1r, n1r, i1r, s2r, n2r, i2r):
    f32 = jnp.float32
    r = lax.broadcasted_iota(jnp.int32, (_D, _D), 0)
    cidx = lax.broadcasted_iota(jnp.int32, (_D, _D), 1)
    ltri = (cidx <= r).astype(f32)
    utri = (r <= cidx).astype(f32)

    def mm(a, b):
        return lax.dot_general(
            a, b, (((1,), (0,)), ((), ())),
            precision=lax.Precision.HIGHEST,
            preferred_element_type=f32)

    a = _logistic(d0r[0])
    s = mm(ltri, a.reshape(_D, -1)).reshape(a.shape)
    first = s[0:1]
    last = s[_D - 1:_D]
    n = (_D - 1.0) * (s - first) / (last - first + 1e-7)
    s0r[...] = (s - 1.0)[None]
    n0r[...] = n[None]
    base = lax.broadcasted_iota(jnp.int32, a.shape, 0).astype(f32)
    i0r[...] = (2.0 * base - n)[None]

    a = _logistic(d1r[0])
    base = lax.broadcasted_iota(jnp.int32, (_D, _D), 0).astype(f32)
    for i in range(a.shape[0]):
        s = mm(ltri, a[i])
        first = s[0:1, :]
        last = s[_D - 1:_D, :]
        n = (_D - 1.0) * (s - first) / (last - first + 1e-7)
        s1r[0, i] = s - 1.0
        n1r[0, i] = n
        i1r[0, i] = 2.0 * base - n

    a = _logistic(d2r[0])
    base = lax.broadcasted_iota(jnp.int32, (_D, _D), 1).astype(f32)
    for i in range(a.shape[0]):
        s = mm(a[i], utri)
        first = s[:, 0:1]
        last = s[:, _D - 1:_D]
        n = (_D - 1.0) * (s - first) / (last - first + 1e-7)
        s2r[0, i] = s - 1.0
        n2r[0, i] = n
        i2r[0, i] = 2.0 * base - n


def _grids(d0, d1, d2, interpret=False):
    xspec = pl.BlockSpec((1, _D, 16, _D), lambda b, j: (b, 0, j, 0))
    yspec = pl.BlockSpec((1, 16, _D, _D), lambda b, j: (b, j, 0, 0))
    shp = jax.ShapeDtypeStruct((_B, _D, _D, _D), jnp.float32)
    return pl.pallas_call(
        _grids_body,
        grid=(_B, _D // 16),
        in_specs=[xspec, yspec, yspec],
        out_specs=[xspec, xspec, xspec,
                   yspec, yspec, yspec,
                   yspec, yspec, yspec],
        out_shape=[shp] * 9,
        interpret=interpret,
    )(d0, d1, d2)


_TAB = 64 * _D2


def _pp_body(cur_r, o_r):
    a = cur_r[0].reshape(8, 2, _D, _D)
    lo = lax.bitcast_convert_type(
        a[:, 0].astype(jnp.bfloat16), jnp.uint16).astype(jnp.uint32)
    hi = lax.bitcast_convert_type(
        a[:, 1].astype(jnp.bfloat16), jnp.uint16).astype(jnp.uint32)
    w = lax.bitcast_convert_type(lo | (hi << 16), jnp.int32)
    o_r[...] = w.reshape(8 * _D, _D)[None]


def _pppack(imt):
    return pl.pallas_call(
        _pp_body,
        grid=(_B, _D // 16),
        in_specs=[pl.BlockSpec((1, 16, _D, _D), lambda b, z: (b, z, 0, 0))],
        out_specs=pl.BlockSpec((1, 8 * _D, _D), lambda b, z: (b, z, 0)),
        out_shape=jax.ShapeDtypeStruct((_B, 64 * _D, _D), jnp.int32),
    )(imt).reshape(_B * _TAB)


_NW = 32
_NPW = _N // _NW
_CK = 1024
_NCHUNK = _NPW // _CK
_ROWS = _CK // _D
_STG = _TAB // 16
_STH = 8192


def _resample_body(pp, cx, cy, cz, out, *sc):
    seta = sc[0:24]
    setb = sc[24:48]
    stb = sc[48]
    tab = sc[49]
    sem_a, sem_b, sem_oa, sem_ob = sc[50:54]
    cid = lax.axis_index("c")
    sid = lax.axis_index("s")
    base_pt = (cid * 16 + sid) * _NPW

    for h in range(_STG // _STH):
        soff = sid * _STG + h * _STH
        pltpu.sync_copy(pp.at[pl.ds(cid * _TAB + soff, _STH)], stb)
        pltpu.sync_copy(stb, tab.at[pl.ds(soff, _STH)])
    plsc.subcore_barrier()

    def mkset(bufs, gsem, osem):
        cxb, cyb, czb, xdb, ydb, zdb, pob, outb = bufs[0:8]
        ii = bufs[8:16]
        vv = bufs[16:24]

        def fetch_prep_fire(t):
            off = base_pt + t * _CK
            c1 = pltpu.async_copy(cx.at[pl.ds(off, _CK)], cxb, gsem)
            c2 = pltpu.async_copy(cy.at[pl.ds(off, _CK)], cyb, gsem)
            c3 = pltpu.async_copy(cz.at[pl.ds(off, _CK)], czb, gsem)
            c1.wait()
            c2.wait()
            c3.wait()

            def prep(i, _):
                sl = pl.ds(i * 16, 16)
                x = jnp.clip(cxb[sl], 0.0, _D - 1.0)
                y = jnp.clip(cyb[sl], 0.0, _D - 1.0)
                z = jnp.clip(czb[sl], 0.0, _D - 1.0)
                x0 = jnp.minimum(x.astype(jnp.int32), _D - 2)
                y0 = jnp.minimum(y.astype(jnp.int32), _D - 2)
                z0 = jnp.minimum(z.astype(jnp.int32), _D - 2)
                xdb[sl] = x - x0.astype(jnp.float32)
                ydb[sl] = y - y0.astype(jnp.float32)
                zdb[sl] = z - z0.astype(jnp.float32)
                podd = z0 & 1
                pob[sl] = podd
                v = (z0 >> 1) * _D2 + y0 * _D + x0
                vb = v + podd * _D2
                ii[0][sl] = v
                ii[1][sl] = vb
                ii[2][sl] = v + 1
                ii[3][sl] = vb + 1
                ii[4][sl] = v + _D
                ii[5][sl] = vb + _D
                ii[6][sl] = v + _D + 1
                ii[7][sl] = vb + _D + 1
                return 0

            lax.fori_loop(0, _CK // 16, prep, 0)
            for g in range(8):
                pltpu.async_copy(tab.at[ii[g]], vv[g], gsem)

        def blend_out(t):
            for g in range(8):
                pltpu.make_async_copy(
                    pp.at[pl.ds(0, _CK)], vv[g], gsem).wait()

            @pl.when(t >= 2)
            def _():
                pltpu.make_async_copy(
                    out.at[pl.ds(base_pt, _CK)], outb, osem).wait()

            def blend(i, _):
                sl = pl.ds(i * 16, 16)
                xd = xdb[sl]
                yd = ydb[sl]
                zd = zdb[sl]
                odd = pob[sl] == 1

                def zmix(va, vbb):
                    wa = va[sl]
                    wb = vbb[sl]
                    lo1 = lax.bitcast_convert_type(wa << 16, jnp.float32)
                    hi1 = lax.bitcast_convert_type(wa & jnp.int32(-65536),
                                                   jnp.float32)
                    lo2 = lax.bitcast_convert_type(wb << 16, jnp.float32)
                    hi2 = lax.bitcast_convert_type(wb & jnp.int32(-65536),
                                                   jnp.float32)
                    vz0 = jnp.where(odd, hi1, lo1)
                    vz1 = jnp.where(odd, lo2, hi2)
                    return vz0 + zd * (vz1 - vz0)

                c00 = zmix(vv[0], vv[1])
                c01 = zmix(vv[2], vv[3])
                c10 = zmix(vv[4], vv[5])
                c11 = zmix(vv[6], vv[7])
                r0 = c00 + xd * (c01 - c00)
                r1 = c10 + xd * (c11 - c10)
                outb[sl] = r0 + yd * (r1 - r0)
                return 0

            lax.fori_loop(0, _CK // 16, blend, 0)
            pltpu.async_copy(outb, out.at[pl.ds(base_pt + t * _CK, _CK)],
                             osem)

        def drain_out():
            pltpu.make_async_copy(
                out.at[pl.ds(base_pt, _CK)], outb, osem).wait()

        return fetch_prep_fire, blend_out, drain_out

    fpf_a, blo_a, dr_a = mkset(seta, sem_a, sem_oa)
    fpf_b, blo_b, dr_b = mkset(setb, sem_b, sem_ob)

    fpf_a(0)

    def pair(u, _):
        t0 = u * 2
        fpf_b(t0 + 1)
        blo_a(t0)
        fpf_a(t0 + 2)
        blo_b(t0 + 1)
        return 0

    lax.fori_loop(0, _NCHUNK // 2 - 1, pair, 0)
    fpf_b(_NCHUNK - 1)
    blo_a(_NCHUNK - 2)
    blo_b(_NCHUNK - 1)
    dr_a()
    dr_b()


@functools.partial(jax.jit, static_argnames=("interpret",))
def _resample(pp, cx, cy, cz, interpret=False):
    mesh = plsc.VectorSubcoreMesh(
        core_axis_name="c", subcore_axis_name="s", num_cores=2)
    bufset = ([pltpu.VMEM((_CK,), jnp.float32)] * 6
              + [pltpu.VMEM((_CK,), jnp.int32)]
              + [pltpu.VMEM((_CK,), jnp.float32)]
              + [pltpu.VMEM((_CK,), jnp.int32)] * 16)
    return pl.kernel(
        _resample_body,
        out_type=jax.ShapeDtypeStruct((_N,), jnp.float32),
        mesh=mesh,
        scratch_types=(
            bufset + bufset + [
                pltpu.VMEM((_STH,), jnp.int32),
                pltpu.VMEM_SHARED((_TAB,), jnp.int32),
                pltpu.SemaphoreType.DMA,
                pltpu.SemaphoreType.DMA,
                pltpu.SemaphoreType.DMA,
                pltpu.SemaphoreType.DMA,
            ]),
        interpret=interpret,
    )(pp, cx, cy, cz)


def kernel(mov, ref, defgrad):
    d0 = defgrad[..., 0]
    d1 = defgrad[..., 1]
    d2 = defgrad[..., 2]
    s0, n0, i0, s1, n1, i1, s2, n2, i2 = _grids(d0, d1, d2)

    norm = jnp.stack([n0, n1, n2], axis=-1)
    inverse = jnp.stack([i0, i1, i2], axis=-1)

    mov_t = jnp.transpose(mov.reshape(_B, _D, _D, _D), (0, 3, 2, 1))
    ref_t = jnp.transpose(ref.reshape(_B, _D, _D, _D), (0, 3, 2, 1))
    mov_pp = _pppack(mov_t)
    ref_pp = _pppack(ref_t)

    mov_def = _resample(mov_pp, s0.reshape(-1), s1.reshape(-1),
                        s2.reshape(-1))
    ref_def = _resample(ref_pp, i0.reshape(-1), i1.reshape(-1),
                        i2.reshape(-1))

    out_shape = (_B, _D, _D, _D, 1)
    return (mov_def.reshape(out_shape), ref_def.reshape(out_shape),
            norm, inverse)
